# Initial kernel scaffold; baseline (speedup 1.0000x reference)
import jax
import jax.numpy as jnp
from jax.experimental import pallas as pl


def kernel(x, W):
    raise NotImplementedError("write your pallas kernel here")


# speedup vs baseline: 1.4020x; 1.4020x over previous
import functools

import numpy as np
import jax
import jax.numpy as jnp
from jax import lax
from jax.experimental import pallas as pl
from jax.experimental.pallas import tpu as pltpu
from jax.experimental.pallas import tpu_sc as plsc

_FIELD_DIMS = [40000] * 26
_NF = 26
_EMB = 32
_BATCH = 16384
_ROWS = _BATCH * _NF
_NW = 32
_BPW = _ROWS // _NW
_CHUNK = 128
_NCH = _BPW // _CHUNK
_LANES = 16
_VPC = _CHUNK // _LANES

_FDIM = _FIELD_DIMS[0]

_mesh = plsc.VectorSubcoreMesh(core_axis_name="c", subcore_axis_name="s")


@functools.partial(
    pl.kernel,
    mesh=_mesh,
    out_type=jax.ShapeDtypeStruct((_ROWS, _EMB), jnp.float32),
    compiler_params=pltpu.CompilerParams(use_tc_tiling_on_sc=False),
    scratch_types=[
        pltpu.VMEM((_NCH, _CHUNK), jnp.int32),
        pltpu.VMEM((_CHUNK, _EMB), jnp.float32),
        pltpu.SemaphoreType.DMA,
    ],
)
def _emb_gather(x_hbm, w_hbm, out_hbm, idx_v, rows_v, gsem):
    cid = lax.axis_index("c")
    sid = lax.axis_index("s")
    wid = sid * 2 + cid
    row0 = wid * _BPW

    pltpu.sync_copy(x_hbm.at[pl.ds(wid * _NCH, _NCH)], idx_v)

    def _fix(i, _):
        g0 = row0 + i * _LANES
        f = lax.rem(g0 + lax.iota(jnp.int32, _LANES), _NF)
        j = i // _VPC
        k = (i % _VPC) * _LANES
        idx_v[j, pl.ds(k, _LANES)] = idx_v[j, pl.ds(k, _LANES)] + f * _FDIM
        return ()

    lax.fori_loop(0, _NCH * _VPC, _fix, (), unroll=2)

    def _go(j, _):
        pltpu.async_copy(w_hbm.at[idx_v.at[j]], rows_v, gsem).wait()
        pltpu.sync_copy(rows_v, out_hbm.at[pl.ds(row0 + j * _CHUNK, _CHUNK)])
        return ()

    lax.fori_loop(0, _NCH, _go, ())


def kernel(x, W):
    x2 = x.astype(jnp.int32).reshape(_ROWS // _CHUNK, _CHUNK)
    out = _emb_gather(x2, W.astype(jnp.float32))
    return out.reshape(_BATCH, _NF, _EMB)


# speedup vs baseline: 1.5362x; 1.0957x over previous
import functools

import numpy as np
import jax
import jax.numpy as jnp
from jax import lax
from jax.experimental import pallas as pl
from jax.experimental.pallas import tpu as pltpu
from jax.experimental.pallas import tpu_sc as plsc

_FIELD_DIMS = [40000] * 26
_NF = 26
_EMB = 32
_BATCH = 16384
_ROWS = _BATCH * _NF
_NW = 32
_BPW = _ROWS // _NW
_CHUNK = 128
_NCH = _BPW // _CHUNK
_LANES = 16
_VPC = _CHUNK // _LANES

_FDIM = _FIELD_DIMS[0]
_TOTAL = _NF * _FDIM
_NB = 8
_NR = _NCH // _NB

_mesh = plsc.VectorSubcoreMesh(core_axis_name="c", subcore_axis_name="s")


@functools.partial(
    pl.kernel,
    mesh=_mesh,
    out_type=jax.ShapeDtypeStruct((_ROWS, _EMB), jnp.float32),
    compiler_params=pltpu.CompilerParams(use_tc_tiling_on_sc=False),
    scratch_types=[
        pltpu.VMEM((_NCH, _CHUNK), jnp.int32),
        pltpu.VMEM((_NB, _CHUNK, _EMB), jnp.float32),
        [pltpu.SemaphoreType.DMA] * _NB,
        [pltpu.SemaphoreType.DMA] * _NB,
    ],
)
def _emb_gather(x_hbm, w_hbm, out_hbm, idx_v, bufs, gsems, ssems):
    cid = lax.axis_index("c")
    sid = lax.axis_index("s")
    wid = sid * 2 + cid
    row0 = wid * _BPW

    pltpu.sync_copy(x_hbm.at[pl.ds(wid * _NCH, _NCH)], idx_v)

    off0 = lax.rem(row0 + lax.iota(jnp.int32, _LANES), _NF) * _FDIM
    step = (_LANES % _NF) * _FDIM

    def _fix(j, off):
        for t in range(_VPC):
            s = pl.ds(t * _LANES, _LANES)
            idx_v[j, s] = idx_v[j, s] + off
            off = off + step
            off = jnp.where(off >= _TOTAL, off - _TOTAL, off)
        return off

    lax.fori_loop(0, _NCH, _fix, off0)

    def _round(r, _):
        handles = []
        for t in range(_NB):
            @pl.when(r > 0)
            def _drain(t=t):
                pltpu.make_async_copy(
                    bufs.at[t], out_hbm.at[pl.ds(row0, _CHUNK)], ssems[t]
                ).wait()
            j = r * _NB + t
            handles.append(
                pltpu.async_copy(w_hbm.at[idx_v.at[j]], bufs.at[t], gsems[t]))
        for t in range(_NB):
            handles[t].wait()
            j = r * _NB + t
            pltpu.async_copy(
                bufs.at[t], out_hbm.at[pl.ds(row0 + j * _CHUNK, _CHUNK)],
                ssems[t])
        return ()

    lax.fori_loop(0, _NR, _round, ())

    for t in range(_NB):
        pltpu.make_async_copy(
            bufs.at[t], out_hbm.at[pl.ds(row0, _CHUNK)], ssems[t]).wait()


def kernel(x, W):
    x2 = x.astype(jnp.int32).reshape(_ROWS // _CHUNK, _CHUNK)
    out = _emb_gather(x2, W.astype(jnp.float32))
    return out.reshape(_BATCH, _NF, _EMB)
